# baseline (speedup 1.0000x reference)
import jax
import jax.numpy as jnp
from jax import lax
from jax.experimental import pallas as pl
from jax.experimental.pallas import tpu as pltpu
from jax.experimental.pallas import tpu_sc as plsc

VOCAB = 1000000
DIM = 32
B = 16384
L = 20
BL = B * L

_info = plsc.get_sparse_core_info()
NC, NS = _info.num_cores, _info.num_subcores
NW = NC * NS
GB = 512
NL2 = L // 2
OUT_WORDS = 2 * L * 4 * 128 * 8 * 128


def _gather_body(idx0_hbm, idx1_hbm, tab0_hbm, tab1_hbm, out_hbm,
                 idx_a, idx_b, rows_a, rows_b, w_a, w_b,
                 si_a, si_b, sg_a, sg_b, sw_a, sw_b):
    wid = lax.axis_index("s") * NC + lax.axis_index("c")
    b0 = wid * GB

    iota = lax.iota(jnp.int32, 16)
    zeros16 = iota * 0

    def idx_off(t, l):
        del t
        return l * B + b0

    def fire_idx(idx_hbm, t, l, buf, sem):
        return pltpu.async_copy(idx_hbm.at[pl.ds(idx_off(t, l), GB)],
                                buf, sem)

    def wait_idx(idx_hbm, t, l, buf, sem):
        pltpu.make_async_copy(idx_hbm.at[pl.ds(idx_off(t, l), GB)],
                              buf, sem).wait()

    def fire_gather(tab_hbm, ibuf, rbuf, sem):
        return pltpu.async_copy(tab_hbm.at[ibuf], rbuf, sem)

    def wait_gather(tab_hbm, ibuf, rbuf, sem):
        pltpu.make_async_copy(tab_hbm.at[ibuf], rbuf, sem).wait()

    def transpose(rbuf, wbuf):
        def q_body(q, carry):
            rb = q // 32
            j = (q // 8) % 4
            s = q % 8
            d = 8 * rb + s
            idx_d = zeros16 + d
            u = rb * 4096 + (j * 8 + s) * 128
            for m in range(8):
                idx_bv = iota + (128 * j + 16 * m)
                vec = plsc.load_gather(rbuf, [idx_bv, idx_d])
                wbuf[pl.ds(u + 16 * m, 16)] = vec
            return carry
        lax.fori_loop(0, 128, q_body, 0, unroll=False)

    def out_off(t, l, rb):
        return (((t * L + l) * 4 + rb) * 128 + 4 * wid) * 1024

    def fire_writes(t, l, wbuf, sem):
        for rb in range(4):
            pltpu.async_copy(wbuf.at[pl.ds(rb * 4096, 4096)],
                             out_hbm.at[pl.ds(out_off(t, l, rb), 4096)], sem)

    def wait_writes(wbuf, sem):
        for rb in range(4):
            pltpu.make_async_copy(wbuf.at[pl.ds(rb * 4096, 4096)],
                                  out_hbm.at[pl.ds(0, 4096)], sem).wait()

    for t, (idx_hbm, tab_hbm) in enumerate(((idx0_hbm, tab0_hbm),
                                            (idx1_hbm, tab1_hbm))):
        fire_idx(idx_hbm, t, 0, idx_a, si_a)
        fire_idx(idx_hbm, t, 1, idx_b, si_b)
        wait_idx(idx_hbm, t, 0, idx_a, si_a)
        fire_gather(tab_hbm, idx_a, rows_a, sg_a)

        def l_body(k, carry):
            la = 2 * k
            lb = la + 1
            wait_idx(idx_hbm, t, lb, idx_b, si_b)
            fire_gather(tab_hbm, idx_b, rows_b, sg_b)

            wait_gather(tab_hbm, idx_a, rows_a, sg_a)

            @pl.when(k > 0)
            def _():
                wait_writes(w_a, sw_a)
            transpose(rows_a, w_a)
            fire_writes(t, la, w_a, sw_a)

            @pl.when(k + 1 < NL2)
            def _():
                fire_idx(idx_hbm, t, la + 2, idx_a, si_a)

            wait_gather(tab_hbm, idx_b, rows_b, sg_b)

            @pl.when(k > 0)
            def _():
                wait_writes(w_b, sw_b)
            transpose(rows_b, w_b)
            fire_writes(t, lb, w_b, sw_b)

            @pl.when(k + 1 < NL2)
            def _():
                wait_idx(idx_hbm, t, la + 2, idx_a, si_a)
                fire_gather(tab_hbm, idx_a, rows_a, sg_a)
                fire_idx(idx_hbm, t, lb + 2, idx_b, si_b)
            return carry

        lax.fori_loop(0, NL2, l_body, 0, unroll=False)
        wait_writes(w_a, sw_a)
        wait_writes(w_b, sw_b)


_mesh = plsc.VectorSubcoreMesh(core_axis_name="c", subcore_axis_name="s")

_sc_gather = pl.kernel(
    _gather_body,
    out_type=jax.ShapeDtypeStruct((OUT_WORDS,), jnp.float32),
    mesh=_mesh,
    scratch_types=[
        pltpu.VMEM((GB,), jnp.int32),
        pltpu.VMEM((GB,), jnp.int32),
        pltpu.VMEM((GB, DIM), jnp.float32),
        pltpu.VMEM((GB, DIM), jnp.float32),
        pltpu.VMEM((16384,), jnp.float32),
        pltpu.VMEM((16384,), jnp.float32),
        pltpu.SemaphoreType.DMA,
        pltpu.SemaphoreType.DMA,
        pltpu.SemaphoreType.DMA,
        pltpu.SemaphoreType.DMA,
        pltpu.SemaphoreType.DMA,
        pltpu.SemaphoreType.DMA,
    ],
    compiler_params=pltpu.CompilerParams(use_tc_tiling_on_sc=False,
                                         needs_layout_passes=False),
)


@jax.jit
def kernel(idx0, idx1, table0, table1):
    i0 = idx0.T.reshape(BL).astype(jnp.int32)
    i1 = idx1.T.reshape(BL).astype(jnp.int32)
    out = _sc_gather(i0, i1, table0, table1)
    y = out.reshape(2, L, 4, 128, 8, 128).transpose(0, 3, 5, 1, 2, 4)
    return y.reshape(2, B, L, DIM)
